# Initial kernel scaffold; baseline (speedup 1.0000x reference)
import jax
import jax.numpy as jnp
from jax.experimental import pallas as pl


def kernel(x, edge_attr, a, b, gamma1, gamma2, bias, W1, b1, W2, b2, edge_index):
    raise NotImplementedError("write your pallas kernel here")


# speedup vs baseline: 2.7904x; 2.7904x over previous
import functools

import jax
import jax.numpy as jnp
from jax.experimental import pallas as pl

_N = 50000
_E = 1600000
_H = 20
_EH = 64
_EO = 10
_THRESH = 1.0


def _eac_body(t_ref, w1_ref, b1_ref, w2_ref, b2_ref, out_ref):
    t = t_ref[...]
    w1 = w1_ref[...]
    h1 = jnp.maximum(t * w1[0][None, :] + b1_ref[...][None, :], 0.0)
    mlp = jnp.dot(h1, w2_ref[...], preferred_element_type=jnp.float32)
    mlp = mlp + b2_ref[...][None, :]
    idx = jnp.clip((t[:, 0] / (_THRESH / 10.0)).astype(jnp.int32), 0, 9)
    oh = (idx[:, None] == jax.lax.broadcasted_iota(jnp.int32, (1, _EO), 1)).astype(
        jnp.float32
    )
    out_ref[...] = jnp.concatenate([oh, mlp], axis=1)


def _eac_tc(edge_attr, W1, b1, W2, b2):
    BE = 12800
    grid = _E // BE
    return pl.pallas_call(
        _eac_body,
        grid=(grid,),
        in_specs=[
            pl.BlockSpec((BE, 1), lambda i: (i, 0)),
            pl.BlockSpec((1, _EH), lambda i: (0, 0)),
            pl.BlockSpec((_EH,), lambda i: (0,)),
            pl.BlockSpec((_EH, _EO), lambda i: (0, 0)),
            pl.BlockSpec((_EO,), lambda i: (0,)),
        ],
        out_specs=pl.BlockSpec((BE, 2 * _EO), lambda i: (i, 0)),
        out_shape=jax.ShapeDtypeStruct((_E, 2 * _EO), jnp.float32),
    )(edge_attr, W1, b1, W2, b2)


def kernel(x, edge_attr, a, b, gamma1, gamma2, bias, W1, b1, W2, b2, edge_index):
    x0 = x[:, 0, :]
    src = edge_index[0]
    dst = edge_index[1]
    eac = _eac_tc(edge_attr, W1, b1, W2, b2)
    hs = x0[src]
    hd = x0[dst]
    rho = jnp.abs(a * hs - (1.0 - a) * hd) ** b
    denom = jax.ops.segment_sum(eac, src, num_segments=_N)
    s1 = jax.ops.segment_sum(rho * eac, src, num_segments=_N)
    s0 = jax.ops.segment_sum(rho, src, num_segments=_N)
    sf = jnp.where(denom != 0, s1 / jnp.where(denom != 0, denom, 1.0), 0.01 * s0)
    out0 = jax.nn.sigmoid(x0 @ gamma1.T + sf @ gamma2.T + bias)
    return jnp.stack([out0, sf], axis=1)


# speedup vs baseline: 4.4567x; 1.5972x over previous
import functools

import jax
import jax.numpy as jnp
from jax import lax
from jax.experimental import pallas as pl
from jax.experimental.pallas import tpu as pltpu
from jax.experimental.pallas import tpu_sc as plsc

_N = 50000
_E = 1600000
_H = 20
_EH = 64
_EO = 10
_THRESH = 1.0


def _eac_body(t_ref, w1_ref, b1_ref, w2_ref, b2_ref, out_ref):
    t = t_ref[...]
    w1 = w1_ref[...]
    h1 = jnp.maximum(t * w1[0][None, :] + b1_ref[...][None, :], 0.0)
    mlp = jnp.dot(h1, w2_ref[...], preferred_element_type=jnp.float32)
    mlp = mlp + b2_ref[...][None, :]
    idx = jnp.clip((t[:, 0] / (_THRESH / 10.0)).astype(jnp.int32), 0, 9)
    oh = (idx[:, None] == jax.lax.broadcasted_iota(jnp.int32, (1, _EO), 1)).astype(
        jnp.float32
    )
    out_ref[...] = jnp.concatenate([oh, mlp], axis=1)


def _eac_tc(edge_attr, W1, b1, W2, b2):
    BE = 12800
    grid = _E // BE
    return pl.pallas_call(
        _eac_body,
        grid=(grid,),
        in_specs=[
            pl.BlockSpec((BE, 1), lambda i: (i, 0)),
            pl.BlockSpec((1, _EH), lambda i: (0, 0)),
            pl.BlockSpec((_EH,), lambda i: (0,)),
            pl.BlockSpec((_EH, _EO), lambda i: (0, 0)),
            pl.BlockSpec((_EO,), lambda i: (0,)),
        ],
        out_specs=pl.BlockSpec((BE, 2 * _EO), lambda i: (i, 0)),
        out_shape=jax.ShapeDtypeStruct((_E, 2 * _EO), jnp.float32),
    )(edge_attr, W1, b1, W2, b2)


_W = 2000
_IW = 40
_EPT = _E // 16
_NWIN = _EPT // _W
_RPT = _N // 16


def _scatter_body(src_hbm, p_hbm, z_hbm, out_hbm, idx_v, upd_v, acc_sh):
    c = lax.axis_index("c")
    s = lax.axis_index("s")
    r0 = s * _RPT
    for q in range(2):
        slab = 2 * q + c
        pltpu.sync_copy(z_hbm.at[pl.ds(r0, _RPT)], acc_sh.at[pl.ds(r0, _RPT)])
        plsc.subcore_barrier()

        def win(w, carry):
            base = s * _EPT + w * _W
            pltpu.sync_copy(
                src_hbm.at[pl.ds(base // _IW, _W // _IW)], idx_v)
            pltpu.sync_copy(p_hbm.at[slab, pl.ds(base, _W)], upd_v)
            for j in range(_W // _IW):
                pltpu.sync_copy(upd_v.at[pl.ds(j * _IW, _IW)],
                                acc_sh.at[idx_v.at[j, 0]], add=True)
            return carry

        lax.fori_loop(0, _NWIN, win, 0)
        plsc.subcore_barrier()
        pltpu.sync_copy(acc_sh.at[pl.ds(r0, _RPT)],
                        out_hbm.at[slab, pl.ds(r0, _RPT)])
        plsc.subcore_barrier()


@functools.partial(
    pl.kernel,
    out_type=jax.ShapeDtypeStruct((4, _N, 16), jnp.float32),
    mesh=plsc.VectorSubcoreMesh(core_axis_name="c", subcore_axis_name="s"),
    compiler_params=pltpu.CompilerParams(use_tc_tiling_on_sc=False),
    scratch_types=[
        pltpu.VMEM((_W // _IW, 1, _IW), jnp.int32),
        pltpu.VMEM((_W, 16), jnp.float32),
        pltpu.VMEM_SHARED((_N, 16), jnp.float32),
    ],
)
def _scatter_sc(src_hbm, p_hbm, z_hbm, out_hbm, idx_v, upd_v, acc_sh):
    _scatter_body(src_hbm, p_hbm, z_hbm, out_hbm, idx_v, upd_v, acc_sh)


def kernel(x, edge_attr, a, b, gamma1, gamma2, bias, W1, b1, W2, b2, edge_index):
    x0 = x[:, 0, :]
    src = edge_index[0]
    dst = edge_index[1]
    eac = _eac_tc(edge_attr, W1, b1, W2, b2)
    hs = x0[src]
    hd = x0[dst]
    rho = jnp.abs(a * hs - (1.0 - a) * hd) ** b
    rl = rho * eac
    payload = jnp.concatenate([eac, rl, rho, jnp.zeros((_E, 4), jnp.float32)],
                              axis=1)
    p = payload.T.reshape(4, 16, _E).transpose(0, 2, 1)
    src2d = src.reshape(_E // _IW, 1, _IW)
    zeros = jnp.zeros((_N, 16), jnp.float32)
    accs = _scatter_sc(src2d, p, zeros)
    acc = accs.transpose(0, 2, 1).reshape(64, _N).T
    denom = acc[:, :20]
    s1 = acc[:, 20:40]
    s0 = acc[:, 40:60]
    sf = jnp.where(denom != 0, s1 / jnp.where(denom != 0, denom, 1.0), 0.01 * s0)
    out0 = jax.nn.sigmoid(x0 @ gamma1.T + sf @ gamma2.T + bias)
    return jnp.stack([out0, sf], axis=1)


# speedup vs baseline: 15.0039x; 3.3666x over previous
import functools

import jax
import jax.numpy as jnp
from jax import lax
from jax.experimental import pallas as pl
from jax.experimental.pallas import tpu as pltpu
from jax.experimental.pallas import tpu_sc as plsc

_N = 50000
_E = 1600000
_EP = 1638400
_H = 20
_EH = 64
_EO = 10
_THRESH = 1.0

_WG = 1024
_WS = 512
_IW = 128
_EPW = _EP // 32
_EPT = _EP // 16

_mesh = plsc.VectorSubcoreMesh(core_axis_name="c", subcore_axis_name="s")
_sc_params = pltpu.CompilerParams(use_tc_tiling_on_sc=False)


@functools.partial(
    pl.kernel,
    out_type=jax.ShapeDtypeStruct((_EP // 4, 128), jnp.float32),
    mesh=_mesh,
    compiler_params=_sc_params,
    scratch_types=[
        pltpu.VMEM((_WG // _IW, 1, _IW), jnp.int32),
        pltpu.VMEM((_WG // _IW, 1, _IW), jnp.int32),
        pltpu.VMEM((_WG, 32), jnp.float32),
        pltpu.VMEM((_WG, 32), jnp.float32),
        pltpu.VMEM((_WG // 4, 128), jnp.float32),
        pltpu.VMEM((16,), jnp.float32),
    ],
)
def _gather_sc(x0p_hbm, src_hbm, dst_hbm, av_hbm, au_hbm,
               idxs_v, idxd_v, hs_v, hd_v, au_v, av_v):
    wid = lax.axis_index("s") * 2 + lax.axis_index("c")
    pltpu.sync_copy(av_hbm, av_v)
    a_vec = av_v[...]
    na_vec = 1.0 - a_vec

    def win(w, carry):
        base = wid * _EPW + w * _WG
        row0 = base // _IW
        pltpu.sync_copy(src_hbm.at[pl.ds(row0, _WG // _IW)], idxs_v)
        pltpu.sync_copy(dst_hbm.at[pl.ds(row0, _WG // _IW)], idxd_v)
        for j in range(_WG // _IW):
            pltpu.sync_copy(x0p_hbm.at[idxs_v.at[j, 0]],
                            hs_v.at[pl.ds(j * _IW, _IW)])
            pltpu.sync_copy(x0p_hbm.at[idxd_v.at[j, 0]],
                            hd_v.at[pl.ds(j * _IW, _IW)])

        def rows(r, c2):
            for rr in range(4):
                e = 4 * r + rr
                for k in range(2):
                    u = (a_vec * hs_v[e, pl.ds(16 * k, 16)]
                         - na_vec * hd_v[e, pl.ds(16 * k, 16)])
                    au_v[r, pl.ds(rr * 32 + 16 * k, 16)] = jnp.abs(u)
            return c2

        lax.fori_loop(0, _WG // 4, rows, 0)
        pltpu.sync_copy(au_v, au_hbm.at[pl.ds(base // 4, _WG // 4)])
        return carry

    lax.fori_loop(0, _EPW // _WG, win, 0)


_BE = 12800
_BR = _BE // 4


def _payload_body(t4_ref, au_ref, b_ref, w1r_ref, b1r_ref, w2p_ref, b2p_ref,
                  out_ref):
    i = pl.program_id(0)
    t4 = t4_ref[...]
    t32 = jnp.broadcast_to(t4[:, :, None], (_BR, 4, 32)).reshape(_BR, 128)
    t256 = jnp.broadcast_to(t4[:, :, None], (_BR, 4, 64)).reshape(_BR, 256)
    feat = jnp.maximum(t256 * w1r_ref[...][None, :] + b1r_ref[...][None, :],
                       0.0)
    mlp_p = jnp.dot(feat, w2p_ref[...], preferred_element_type=jnp.float32)
    mlp_p = mlp_p + b2p_ref[...][None, :]
    lanec = lax.broadcasted_iota(jnp.int32, (_BR, 128), 1) % 32
    bidx = jnp.clip((t32 / (_THRESH / 10.0)).astype(jnp.int32), 0, 9)
    oh = (lanec == bidx).astype(jnp.float32)
    eac_p = oh + mlp_p
    rho_p = au_ref[...] ** b_ref[0, 0]
    live = i < _E // _BE
    eac_p = jnp.where(live, eac_p, 0.0)
    rho_p = jnp.where(live, rho_p, 0.0)
    out_ref[0] = eac_p
    out_ref[1] = rho_p * eac_p
    out_ref[2] = rho_p


def _payload_tc(t4, au, b, W1r, b1r, W2P, b2P):
    return pl.pallas_call(
        _payload_body,
        grid=(_EP // _BE,),
        in_specs=[
            pl.BlockSpec((_BR, 4), lambda i: (i, 0)),
            pl.BlockSpec((_BR, 128), lambda i: (i, 0)),
            pl.BlockSpec((1, 1), lambda i: (0, 0)),
            pl.BlockSpec((256,), lambda i: (0,)),
            pl.BlockSpec((256,), lambda i: (0,)),
            pl.BlockSpec((256, 128), lambda i: (0, 0)),
            pl.BlockSpec((128,), lambda i: (0,)),
        ],
        out_specs=pl.BlockSpec((3, _BR, 128), lambda i: (0, i, 0)),
        out_shape=jax.ShapeDtypeStruct((3, _EP // 4, 128), jnp.float32),
    )(t4, au, b, W1r, b1r, W2P, b2P)


_RPT = _N // 16


@functools.partial(
    pl.kernel,
    out_type=jax.ShapeDtypeStruct((4, _N, 32), jnp.float32),
    mesh=_mesh,
    compiler_params=_sc_params,
    scratch_types=[
        pltpu.VMEM((_WS // _IW, 1, _IW), jnp.int32),
        pltpu.VMEM((_WS, 32), jnp.float32),
        pltpu.VMEM_SHARED((_N, 32), jnp.float32),
    ],
)
def _scatter_sc(src_hbm, p_hbm, z_hbm, out_hbm, idx_v, upd_v, acc_sh):
    c = lax.axis_index("c")
    s = lax.axis_index("s")
    r0 = s * _RPT

    def run_phase(slab, out_slab, ebase, ecount):
        pltpu.sync_copy(z_hbm.at[pl.ds(r0, _RPT)], acc_sh.at[pl.ds(r0, _RPT)])
        plsc.subcore_barrier()

        def win(w, carry):
            base = ebase + s * ecount + w * _WS
            pltpu.sync_copy(src_hbm.at[pl.ds(base // _IW, _WS // _IW)], idx_v)
            pltpu.sync_copy(p_hbm.at[slab, pl.ds(base, _WS)], upd_v)
            for j in range(_WS // _IW):
                pltpu.sync_copy(upd_v.at[pl.ds(j * _IW, _IW)],
                                acc_sh.at[idx_v.at[j, 0]], add=True)
            return carry

        lax.fori_loop(0, ecount // _WS, win, 0)
        plsc.subcore_barrier()
        pltpu.sync_copy(acc_sh.at[pl.ds(r0, _RPT)],
                        out_hbm.at[out_slab, pl.ds(r0, _RPT)])
        plsc.subcore_barrier()

    run_phase(c, c, 0, _EPT)
    run_phase(2, 2 + c, c * (_EP // 2), _EP // 32)


def kernel(x, edge_attr, a, b, gamma1, gamma2, bias, W1, b1, W2, b2, edge_index):
    x0 = x[:, 0, :]
    x0p = jnp.pad(x0, ((0, 0), (0, 32 - _H)))
    pad = _EP - _E
    srcp = jnp.concatenate([edge_index[0], jnp.zeros((pad,), jnp.int32)])
    dstp = jnp.concatenate([edge_index[1], jnp.zeros((pad,), jnp.int32)])
    eap = jnp.concatenate([edge_attr[:, 0], jnp.zeros((pad,), jnp.float32)])
    src3 = srcp.reshape(_EP // _IW, 1, _IW)
    dst3 = dstp.reshape(_EP // _IW, 1, _IW)
    av = jnp.full((16,), a[0], jnp.float32)

    W1r = jnp.tile(W1[0], 4)
    b1r = jnp.tile(b1, 4)
    lane = jnp.arange(128)
    hid = jnp.arange(256)
    col = lane % 32
    grp = lane // 32
    sel = (hid[:, None] // _EH == grp[None, :]) & (col[None, :] >= 10) & (
        col[None, :] < 20)
    W2P = jnp.where(sel, W2[hid % _EH][:, jnp.clip(col - 10, 0, 9)], 0.0)
    b2P = jnp.where((col >= 10) & (col < 20), b2[jnp.clip(col - 10, 0, 9)], 0.0)

    au = _gather_sc(x0p, src3, dst3, av)
    t4 = eap.reshape(_EP // 4, 4)
    p = _payload_tc(t4, au, b.reshape(1, 1), W1r, b1r, W2P, b2P)
    p = p.reshape(3, _EP, 32)
    zeros = jnp.zeros((_N, 32), jnp.float32)
    accs = _scatter_sc(src3, p, zeros)
    denom = accs[0, :, :_H]
    s1 = accs[1, :, :_H]
    s0 = (accs[2] + accs[3])[:, :_H]
    sf = jnp.where(denom != 0, s1 / jnp.where(denom != 0, denom, 1.0), 0.01 * s0)
    out0 = jax.nn.sigmoid(x0 @ gamma1.T + sf @ gamma2.T + bias)
    return jnp.stack([out0, sf], axis=1)


# speedup vs baseline: 16.3589x; 1.0903x over previous
import functools

import jax
import jax.numpy as jnp
from jax import lax
from jax.experimental import pallas as pl
from jax.experimental.pallas import tpu as pltpu
from jax.experimental.pallas import tpu_sc as plsc

_N = 50000
_E = 1600000
_EP = 1638400
_H = 20
_EH = 64
_EO = 10
_THRESH = 1.0

_WG = 1024
_WS = 512
_IW = 128
_EPW = _EP // 32
_EPT = _EP // 16

_mesh = plsc.VectorSubcoreMesh(core_axis_name="c", subcore_axis_name="s")
_sc_params = pltpu.CompilerParams(use_tc_tiling_on_sc=False)


@functools.partial(
    pl.kernel,
    out_type=jax.ShapeDtypeStruct((_EP // 4, 128), jnp.float32),
    mesh=_mesh,
    compiler_params=_sc_params,
    scratch_types=[
        pltpu.VMEM((_WG // _IW, 1, _IW), jnp.int32),
        pltpu.VMEM((_WG // _IW, 1, _IW), jnp.int32),
        pltpu.VMEM((_WG, 32), jnp.float32),
        pltpu.VMEM((_WG, 32), jnp.float32),
        pltpu.VMEM((_WG // 4, 128), jnp.float32),
        pltpu.VMEM((16,), jnp.float32),
        pltpu.SemaphoreType.DMA,
    ],
)
def _gather_sc(x0p_hbm, src_hbm, dst_hbm, av_hbm, au_hbm,
               idxs_v, idxd_v, hs_v, hd_v, au_v, av_v, gsem):
    wid = lax.axis_index("s") * 2 + lax.axis_index("c")
    pltpu.sync_copy(av_hbm, av_v)
    a_vec = av_v[...]
    na_vec = 1.0 - a_vec

    def win(w, carry):
        base = wid * _EPW + w * _WG
        row0 = base // _IW
        pltpu.sync_copy(src_hbm.at[pl.ds(row0, _WG // _IW)], idxs_v)
        pltpu.sync_copy(dst_hbm.at[pl.ds(row0, _WG // _IW)], idxd_v)
        cps = []
        for j in range(_WG // _IW):
            cps.append(pltpu.async_copy(x0p_hbm.at[idxs_v.at[j, 0]],
                                        hs_v.at[pl.ds(j * _IW, _IW)], gsem))
            cps.append(pltpu.async_copy(x0p_hbm.at[idxd_v.at[j, 0]],
                                        hd_v.at[pl.ds(j * _IW, _IW)], gsem))
        for cp in cps:
            cp.wait()

        def rows(r, c2):
            for rr in range(4):
                e = 4 * r + rr
                for k in range(2):
                    u = (a_vec * hs_v[e, pl.ds(16 * k, 16)]
                         - na_vec * hd_v[e, pl.ds(16 * k, 16)])
                    au_v[r, pl.ds(rr * 32 + 16 * k, 16)] = jnp.abs(u)
            return c2

        lax.fori_loop(0, _WG // 4, rows, 0)
        pltpu.sync_copy(au_v, au_hbm.at[pl.ds(base // 4, _WG // 4)])
        return carry

    lax.fori_loop(0, _EPW // _WG, win, 0)


_BE = 12800
_BR = _BE // 4


def _payload_body(t4_ref, au_ref, b_ref, w1r_ref, b1r_ref, w2p_ref, b2p_ref,
                  out_ref):
    i = pl.program_id(0)
    t4 = t4_ref[...]
    t32 = jnp.broadcast_to(t4[:, :, None], (_BR, 4, 32)).reshape(_BR, 128)
    t256 = jnp.broadcast_to(t4[:, :, None], (_BR, 4, 64)).reshape(_BR, 256)
    feat = jnp.maximum(t256 * w1r_ref[...][None, :] + b1r_ref[...][None, :],
                       0.0)
    mlp_p = jnp.dot(feat, w2p_ref[...], preferred_element_type=jnp.float32)
    mlp_p = mlp_p + b2p_ref[...][None, :]
    lanec = lax.broadcasted_iota(jnp.int32, (_BR, 128), 1) % 32
    bidx = jnp.clip((t32 / (_THRESH / 10.0)).astype(jnp.int32), 0, 9)
    oh = (lanec == bidx).astype(jnp.float32)
    eac_p = oh + mlp_p
    rho_p = au_ref[...] ** b_ref[0, 0]
    live = i < _E // _BE
    eac_p = jnp.where(live, eac_p, 0.0)
    rho_p = jnp.where(live, rho_p, 0.0)
    out_ref[0] = eac_p
    out_ref[1] = rho_p * eac_p
    out_ref[2] = rho_p


def _payload_tc(t4, au, b, W1r, b1r, W2P, b2P):
    return pl.pallas_call(
        _payload_body,
        grid=(_EP // _BE,),
        in_specs=[
            pl.BlockSpec((_BR, 4), lambda i: (i, 0)),
            pl.BlockSpec((_BR, 128), lambda i: (i, 0)),
            pl.BlockSpec((1, 1), lambda i: (0, 0)),
            pl.BlockSpec((256,), lambda i: (0,)),
            pl.BlockSpec((256,), lambda i: (0,)),
            pl.BlockSpec((256, 128), lambda i: (0, 0)),
            pl.BlockSpec((128,), lambda i: (0,)),
        ],
        out_specs=pl.BlockSpec((3, _BR, 128), lambda i: (0, i, 0)),
        out_shape=jax.ShapeDtypeStruct((3, _EP // 4, 128), jnp.float32),
    )(t4, au, b, W1r, b1r, W2P, b2P)


_RPT = _N // 16


@functools.partial(
    pl.kernel,
    out_type=jax.ShapeDtypeStruct((4, _N, 32), jnp.float32),
    mesh=_mesh,
    compiler_params=_sc_params,
    scratch_types=[
        pltpu.VMEM((_WS // _IW, 1, _IW), jnp.int32),
        pltpu.VMEM((_WS, 32), jnp.float32),
        pltpu.VMEM_SHARED((_N, 32), jnp.float32),
        pltpu.SemaphoreType.DMA,
    ],
)
def _scatter_sc(src_hbm, p_hbm, z_hbm, out_hbm, idx_v, upd_v, acc_sh, ssem):
    c = lax.axis_index("c")
    s = lax.axis_index("s")
    r0 = s * _RPT

    def run_phase(slab, out_slab, ebase, ecount):
        pltpu.sync_copy(z_hbm.at[pl.ds(r0, _RPT)], acc_sh.at[pl.ds(r0, _RPT)])
        plsc.subcore_barrier()

        def win(w, carry):
            base = ebase + s * ecount + w * _WS
            pltpu.sync_copy(src_hbm.at[pl.ds(base // _IW, _WS // _IW)], idx_v)
            pltpu.sync_copy(p_hbm.at[slab, pl.ds(base, _WS)], upd_v)
            cps = []
            for j in range(_WS // _IW):
                cps.append(pltpu.async_copy(upd_v.at[pl.ds(j * _IW, _IW)],
                                            acc_sh.at[idx_v.at[j, 0]], ssem,
                                            add=True))
            for cp in cps:
                cp.wait()
            return carry

        lax.fori_loop(0, ecount // _WS, win, 0)
        plsc.subcore_barrier()
        pltpu.sync_copy(acc_sh.at[pl.ds(r0, _RPT)],
                        out_hbm.at[out_slab, pl.ds(r0, _RPT)])
        plsc.subcore_barrier()

    run_phase(c, c, 0, _EPT)
    run_phase(2, 2 + c, c * (_EP // 2), _EP // 32)


def kernel(x, edge_attr, a, b, gamma1, gamma2, bias, W1, b1, W2, b2, edge_index):
    x0 = x[:, 0, :]
    x0p = jnp.pad(x0, ((0, 0), (0, 32 - _H)))
    pad = _EP - _E
    srcp = jnp.concatenate([edge_index[0], jnp.zeros((pad,), jnp.int32)])
    dstp = jnp.concatenate([edge_index[1], jnp.zeros((pad,), jnp.int32)])
    eap = jnp.concatenate([edge_attr[:, 0], jnp.zeros((pad,), jnp.float32)])
    src3 = srcp.reshape(_EP // _IW, 1, _IW)
    dst3 = dstp.reshape(_EP // _IW, 1, _IW)
    av = jnp.full((16,), a[0], jnp.float32)

    W1r = jnp.tile(W1[0], 4)
    b1r = jnp.tile(b1, 4)
    lane = jnp.arange(128)
    hid = jnp.arange(256)
    col = lane % 32
    grp = lane // 32
    sel = (hid[:, None] // _EH == grp[None, :]) & (col[None, :] >= 10) & (
        col[None, :] < 20)
    W2P = jnp.where(sel, W2[hid % _EH][:, jnp.clip(col - 10, 0, 9)], 0.0)
    b2P = jnp.where((col >= 10) & (col < 20), b2[jnp.clip(col - 10, 0, 9)], 0.0)

    au = _gather_sc(x0p, src3, dst3, av)
    t4 = eap.reshape(_EP // 4, 4)
    p = _payload_tc(t4, au, b.reshape(1, 1), W1r, b1r, W2P, b2P)
    p = p.reshape(3, _EP, 32)
    zeros = jnp.zeros((_N, 32), jnp.float32)
    accs = _scatter_sc(src3, p, zeros)
    denom = accs[0, :, :_H]
    s1 = accs[1, :, :_H]
    s0 = (accs[2] + accs[3])[:, :_H]
    sf = jnp.where(denom != 0, s1 / jnp.where(denom != 0, denom, 1.0), 0.01 * s0)
    out0 = jax.nn.sigmoid(x0 @ gamma1.T + sf @ gamma2.T + bias)
    return jnp.stack([out0, sf], axis=1)


# speedup vs baseline: 17.3793x; 1.0624x over previous
import functools

import jax
import jax.numpy as jnp
from jax import lax
from jax.experimental import pallas as pl
from jax.experimental.pallas import tpu as pltpu
from jax.experimental.pallas import tpu_sc as plsc

_N = 50000
_E = 1600000
_EP = 1638400
_H = 20
_EH = 64
_EO = 10
_THRESH = 1.0

_WG = 1024
_WS = 512
_IW = 128
_EPW = _EP // 32
_EPT = _EP // 16

_mesh = plsc.VectorSubcoreMesh(core_axis_name="c", subcore_axis_name="s")
_sc_params = pltpu.CompilerParams(use_tc_tiling_on_sc=False)


_WGP = 512
_NWG = _EPW // _WGP


@functools.partial(
    pl.kernel,
    out_type=jax.ShapeDtypeStruct((_EP // 4, 128), jnp.float32),
    mesh=_mesh,
    compiler_params=_sc_params,
    scratch_types=[
        pltpu.VMEM((2, _WGP // _IW, 1, _IW), jnp.int32),
        pltpu.VMEM((2, _WGP // _IW, 1, _IW), jnp.int32),
        pltpu.VMEM((2, _WGP, 32), jnp.float32),
        pltpu.VMEM((2, _WGP, 32), jnp.float32),
        pltpu.VMEM((2, _WGP // 4, 128), jnp.float32),
        pltpu.VMEM((16,), jnp.float32),
        pltpu.SemaphoreType.DMA,
        pltpu.SemaphoreType.DMA,
        pltpu.SemaphoreType.DMA,
        pltpu.SemaphoreType.DMA,
        pltpu.SemaphoreType.DMA,
        pltpu.SemaphoreType.DMA,
    ],
)
def _gather_sc(x0p_hbm, src_hbm, dst_hbm, av_hbm, au_hbm,
               idxs_v, idxd_v, hs_v, hd_v, au_v, av_v,
               si0, si1, sg0, sg1, sa0, sa1):
    si = (si0, si1)
    sg = (sg0, sg1)
    sa = (sa0, sa1)
    wid = lax.axis_index("s") * 2 + lax.axis_index("c")
    pltpu.sync_copy(av_hbm, av_v)
    a_vec = av_v[...]
    na_vec = 1.0 - a_vec

    def idx_slices(w):
        row0 = (wid * _EPW + w * _WGP) // _IW
        return (src_hbm.at[pl.ds(row0, _WGP // _IW)],
                dst_hbm.at[pl.ds(row0, _WGP // _IW)])

    def start_idx(w, b):
        ss, dd = idx_slices(w)
        pltpu.async_copy(ss, idxs_v.at[b], si[b])
        pltpu.async_copy(dd, idxd_v.at[b], si[b])

    def wait_idx(w, b):
        ss, dd = idx_slices(w)
        pltpu.make_async_copy(ss, idxs_v.at[b], si[b]).wait()
        pltpu.make_async_copy(dd, idxd_v.at[b], si[b]).wait()

    def issue_gathers(b):
        for j in range(_WGP // _IW):
            pltpu.async_copy(x0p_hbm.at[idxs_v.at[b, j, 0]],
                             hs_v.at[b, pl.ds(j * _IW, _IW)], sg[b])
            pltpu.async_copy(x0p_hbm.at[idxd_v.at[b, j, 0]],
                             hd_v.at[b, pl.ds(j * _IW, _IW)], sg[b])

    def wait_gathers(b):
        for j in range(_WGP // _IW):
            pltpu.make_async_copy(x0p_hbm.at[idxs_v.at[b, j, 0]],
                                  hs_v.at[b, pl.ds(j * _IW, _IW)],
                                  sg[b]).wait()
            pltpu.make_async_copy(x0p_hbm.at[idxd_v.at[b, j, 0]],
                                  hd_v.at[b, pl.ds(j * _IW, _IW)],
                                  sg[b]).wait()

    def au_slice(w):
        return au_hbm.at[pl.ds((wid * _EPW + w * _WGP) // 4, _WGP // 4)]

    start_idx(0, 0)
    start_idx(1, 1)
    wait_idx(0, 0)
    issue_gathers(0)

    def stage(w, b):
        @pl.when(w + 1 < _NWG)
        def _():
            wait_idx(w + 1, 1 - b)
            issue_gathers(1 - b)

        wait_gathers(b)

        @pl.when(w + 2 < _NWG)
        def _():
            start_idx(w + 2, b)

        @pl.when(w >= 2)
        def _():
            pltpu.make_async_copy(au_v.at[b], au_slice(w - 2), sa[b]).wait()

        def rows(r, c2):
            for rr in range(4):
                e = 4 * r + rr
                for k in range(2):
                    u = (a_vec * hs_v[b, e, pl.ds(16 * k, 16)]
                         - na_vec * hd_v[b, e, pl.ds(16 * k, 16)])
                    au_v[b, r, pl.ds(rr * 32 + 16 * k, 16)] = jnp.abs(u)
            return c2

        lax.fori_loop(0, _WGP // 4, rows, 0)
        pltpu.async_copy(au_v.at[b], au_slice(w), sa[b])

    def body2(h, carry):
        stage(2 * h, 0)
        stage(2 * h + 1, 1)
        return carry

    lax.fori_loop(0, _NWG // 2, body2, 0)
    pltpu.make_async_copy(au_v.at[0], au_slice(_NWG - 2), sa0).wait()
    pltpu.make_async_copy(au_v.at[1], au_slice(_NWG - 1), sa1).wait()


_BE = 12800
_BR = _BE // 4


def _payload_body(t4_ref, au_ref, b_ref, w1r_ref, b1r_ref, w2p_ref, b2p_ref,
                  out_ref):
    i = pl.program_id(0)
    t4 = t4_ref[...]
    t32 = jnp.broadcast_to(t4[:, :, None], (_BR, 4, 32)).reshape(_BR, 128)
    t256 = jnp.broadcast_to(t4[:, :, None], (_BR, 4, 64)).reshape(_BR, 256)
    feat = jnp.maximum(t256 * w1r_ref[...][None, :] + b1r_ref[...][None, :],
                       0.0)
    mlp_p = jnp.dot(feat, w2p_ref[...], preferred_element_type=jnp.float32)
    mlp_p = mlp_p + b2p_ref[...][None, :]
    lanec = lax.broadcasted_iota(jnp.int32, (_BR, 128), 1) % 32
    bidx = jnp.clip((t32 / (_THRESH / 10.0)).astype(jnp.int32), 0, 9)
    oh = (lanec == bidx).astype(jnp.float32)
    eac_p = oh + mlp_p
    rho_p = au_ref[...] ** b_ref[0, 0]
    live = i < _E // _BE
    eac_p = jnp.where(live, eac_p, 0.0)
    rho_p = jnp.where(live, rho_p, 0.0)
    out_ref[0] = eac_p
    out_ref[1] = rho_p * eac_p
    out_ref[2] = rho_p


def _payload_tc(t4, au, b, W1r, b1r, W2P, b2P):
    return pl.pallas_call(
        _payload_body,
        grid=(_EP // _BE,),
        in_specs=[
            pl.BlockSpec((_BR, 4), lambda i: (i, 0)),
            pl.BlockSpec((_BR, 128), lambda i: (i, 0)),
            pl.BlockSpec((1, 1), lambda i: (0, 0)),
            pl.BlockSpec((256,), lambda i: (0,)),
            pl.BlockSpec((256,), lambda i: (0,)),
            pl.BlockSpec((256, 128), lambda i: (0, 0)),
            pl.BlockSpec((128,), lambda i: (0,)),
        ],
        out_specs=pl.BlockSpec((3, _BR, 128), lambda i: (0, i, 0)),
        out_shape=jax.ShapeDtypeStruct((3, _EP // 4, 128), jnp.float32),
    )(t4, au, b, W1r, b1r, W2P, b2P)


_RPT = _N // 16


@functools.partial(
    pl.kernel,
    out_type=jax.ShapeDtypeStruct((4, _N, 32), jnp.float32),
    mesh=_mesh,
    compiler_params=_sc_params,
    scratch_types=[
        pltpu.VMEM((_WS // _IW, 1, _IW), jnp.int32),
        pltpu.VMEM((_WS, 32), jnp.float32),
        pltpu.VMEM_SHARED((_N, 32), jnp.float32),
        pltpu.SemaphoreType.DMA,
    ],
)
def _scatter_sc(src_hbm, p_hbm, z_hbm, out_hbm, idx_v, upd_v, acc_sh, ssem):
    c = lax.axis_index("c")
    s = lax.axis_index("s")
    r0 = s * _RPT

    def run_phase(slab, out_slab, ebase, ecount):
        pltpu.sync_copy(z_hbm.at[pl.ds(r0, _RPT)], acc_sh.at[pl.ds(r0, _RPT)])
        plsc.subcore_barrier()

        def win(w, carry):
            base = ebase + s * ecount + w * _WS
            pltpu.sync_copy(src_hbm.at[pl.ds(base // _IW, _WS // _IW)], idx_v)
            pltpu.sync_copy(p_hbm.at[slab, pl.ds(base, _WS)], upd_v)
            cps = []
            for j in range(_WS // _IW):
                cps.append(pltpu.async_copy(upd_v.at[pl.ds(j * _IW, _IW)],
                                            acc_sh.at[idx_v.at[j, 0]], ssem,
                                            add=True))
            for cp in cps:
                cp.wait()
            return carry

        lax.fori_loop(0, ecount // _WS, win, 0)
        plsc.subcore_barrier()
        pltpu.sync_copy(acc_sh.at[pl.ds(r0, _RPT)],
                        out_hbm.at[out_slab, pl.ds(r0, _RPT)])
        plsc.subcore_barrier()

    run_phase(c, c, 0, _EPT)
    run_phase(2, 2 + c, c * (_EP // 2), _EP // 32)


def kernel(x, edge_attr, a, b, gamma1, gamma2, bias, W1, b1, W2, b2, edge_index):
    x0 = x[:, 0, :]
    x0p = jnp.pad(x0, ((0, 0), (0, 32 - _H)))
    pad = _EP - _E
    srcp = jnp.concatenate([edge_index[0], jnp.zeros((pad,), jnp.int32)])
    dstp = jnp.concatenate([edge_index[1], jnp.zeros((pad,), jnp.int32)])
    eap = jnp.concatenate([edge_attr[:, 0], jnp.zeros((pad,), jnp.float32)])
    src3 = srcp.reshape(_EP // _IW, 1, _IW)
    dst3 = dstp.reshape(_EP // _IW, 1, _IW)
    av = jnp.full((16,), a[0], jnp.float32)

    W1r = jnp.tile(W1[0], 4)
    b1r = jnp.tile(b1, 4)
    lane = jnp.arange(128)
    hid = jnp.arange(256)
    col = lane % 32
    grp = lane // 32
    sel = (hid[:, None] // _EH == grp[None, :]) & (col[None, :] >= 10) & (
        col[None, :] < 20)
    W2P = jnp.where(sel, W2[hid % _EH][:, jnp.clip(col - 10, 0, 9)], 0.0)
    b2P = jnp.where((col >= 10) & (col < 20), b2[jnp.clip(col - 10, 0, 9)], 0.0)

    au = _gather_sc(x0p, src3, dst3, av)
    t4 = eap.reshape(_EP // 4, 4)
    p = _payload_tc(t4, au, b.reshape(1, 1), W1r, b1r, W2P, b2P)
    p = p.reshape(3, _EP, 32)
    zeros = jnp.zeros((_N, 32), jnp.float32)
    accs = _scatter_sc(src3, p, zeros)
    denom = accs[0, :, :_H]
    s1 = accs[1, :, :_H]
    s0 = (accs[2] + accs[3])[:, :_H]
    sf = jnp.where(denom != 0, s1 / jnp.where(denom != 0, denom, 1.0), 0.01 * s0)
    out0 = jax.nn.sigmoid(x0 @ gamma1.T + sf @ gamma2.T + bias)
    return jnp.stack([out0, sf], axis=1)


# speedup vs baseline: 19.1702x; 1.1030x over previous
import functools

import jax
import jax.numpy as jnp
from jax import lax
from jax.experimental import pallas as pl
from jax.experimental.pallas import tpu as pltpu
from jax.experimental.pallas import tpu_sc as plsc

_N = 50000
_E = 1600000
_EP = 1638400
_H = 20
_EH = 64
_EO = 10
_THRESH = 1.0

_WG = 1024
_WS = 256
_IW = 128
_EPW = _EP // 32
_EPT = _EP // 16

_mesh = plsc.VectorSubcoreMesh(core_axis_name="c", subcore_axis_name="s")
_sc_params = pltpu.CompilerParams(use_tc_tiling_on_sc=False)


_WGP = 512
_NWG = _EPW // _WGP


@functools.partial(
    pl.kernel,
    out_type=jax.ShapeDtypeStruct((_EP // 4, 128), jnp.float32),
    mesh=_mesh,
    compiler_params=_sc_params,
    scratch_types=[
        pltpu.VMEM((2, _WGP // _IW, 1, _IW), jnp.int32),
        pltpu.VMEM((2, _WGP // _IW, 1, _IW), jnp.int32),
        pltpu.VMEM((2, _WGP, 32), jnp.float32),
        pltpu.VMEM((2, _WGP, 32), jnp.float32),
        pltpu.VMEM((2, _WGP // 4, 128), jnp.float32),
        pltpu.VMEM((16,), jnp.float32),
        pltpu.SemaphoreType.DMA,
        pltpu.SemaphoreType.DMA,
        pltpu.SemaphoreType.DMA,
        pltpu.SemaphoreType.DMA,
        pltpu.SemaphoreType.DMA,
        pltpu.SemaphoreType.DMA,
    ],
)
def _gather_sc(x0p_hbm, src_hbm, dst_hbm, av_hbm, au_hbm,
               idxs_v, idxd_v, hs_v, hd_v, au_v, av_v,
               si0, si1, sg0, sg1, sa0, sa1):
    si = (si0, si1)
    sg = (sg0, sg1)
    sa = (sa0, sa1)
    wid = lax.axis_index("s") * 2 + lax.axis_index("c")
    pltpu.sync_copy(av_hbm, av_v)
    a_vec = av_v[...]
    na_vec = 1.0 - a_vec

    def idx_slices(w):
        row0 = (wid * _EPW + w * _WGP) // _IW
        return (src_hbm.at[pl.ds(row0, _WGP // _IW)],
                dst_hbm.at[pl.ds(row0, _WGP // _IW)])

    def start_idx(w, b):
        ss, dd = idx_slices(w)
        pltpu.async_copy(ss, idxs_v.at[b], si[b])
        pltpu.async_copy(dd, idxd_v.at[b], si[b])

    def wait_idx(w, b):
        ss, dd = idx_slices(w)
        pltpu.make_async_copy(ss, idxs_v.at[b], si[b]).wait()
        pltpu.make_async_copy(dd, idxd_v.at[b], si[b]).wait()

    def issue_gathers(b):
        for j in range(_WGP // _IW):
            pltpu.async_copy(x0p_hbm.at[idxs_v.at[b, j, 0]],
                             hs_v.at[b, pl.ds(j * _IW, _IW)], sg[b])
            pltpu.async_copy(x0p_hbm.at[idxd_v.at[b, j, 0]],
                             hd_v.at[b, pl.ds(j * _IW, _IW)], sg[b])

    def wait_gathers(b):
        for j in range(_WGP // _IW):
            pltpu.make_async_copy(x0p_hbm.at[idxs_v.at[b, j, 0]],
                                  hs_v.at[b, pl.ds(j * _IW, _IW)],
                                  sg[b]).wait()
            pltpu.make_async_copy(x0p_hbm.at[idxd_v.at[b, j, 0]],
                                  hd_v.at[b, pl.ds(j * _IW, _IW)],
                                  sg[b]).wait()

    def au_slice(w):
        return au_hbm.at[pl.ds((wid * _EPW + w * _WGP) // 4, _WGP // 4)]

    start_idx(0, 0)
    start_idx(1, 1)
    wait_idx(0, 0)
    issue_gathers(0)

    def stage(w, b):
        @pl.when(w + 1 < _NWG)
        def _():
            wait_idx(w + 1, 1 - b)
            issue_gathers(1 - b)

        wait_gathers(b)

        @pl.when(w + 2 < _NWG)
        def _():
            start_idx(w + 2, b)

        @pl.when(w >= 2)
        def _():
            pltpu.make_async_copy(au_v.at[b], au_slice(w - 2), sa[b]).wait()

        def rows(r, c2):
            for rr in range(4):
                e = 4 * r + rr
                for k in range(2):
                    u = (a_vec * hs_v[b, e, pl.ds(16 * k, 16)]
                         - na_vec * hd_v[b, e, pl.ds(16 * k, 16)])
                    au_v[b, r, pl.ds(rr * 32 + 16 * k, 16)] = jnp.abs(u)
            return c2

        lax.fori_loop(0, _WGP // 4, rows, 0)
        pltpu.async_copy(au_v.at[b], au_slice(w), sa[b])

    def body2(h, carry):
        stage(2 * h, 0)
        stage(2 * h + 1, 1)
        return carry

    lax.fori_loop(0, _NWG // 2, body2, 0)
    pltpu.make_async_copy(au_v.at[0], au_slice(_NWG - 2), sa0).wait()
    pltpu.make_async_copy(au_v.at[1], au_slice(_NWG - 1), sa1).wait()


_BE = 12800
_BR = _BE // 4


def _payload_body(t4_ref, au_ref, b_ref, w1r_ref, b1r_ref, w2p_ref, b2p_ref,
                  out_ref):
    i = pl.program_id(0)
    t4 = t4_ref[...]
    t32 = jnp.broadcast_to(t4[:, :, None], (_BR, 4, 32)).reshape(_BR, 128)
    t256 = jnp.broadcast_to(t4[:, :, None], (_BR, 4, 64)).reshape(_BR, 256)
    feat = jnp.maximum(t256 * w1r_ref[...][None, :] + b1r_ref[...][None, :],
                       0.0)
    mlp_p = jnp.dot(feat, w2p_ref[...], preferred_element_type=jnp.float32)
    mlp_p = mlp_p + b2p_ref[...][None, :]
    lanec = lax.broadcasted_iota(jnp.int32, (_BR, 128), 1) % 32
    bidx = jnp.clip((t32 / (_THRESH / 10.0)).astype(jnp.int32), 0, 9)
    oh = (lanec == bidx).astype(jnp.float32)
    eac_p = oh + mlp_p
    rho_p = au_ref[...] ** b_ref[0, 0]
    live = i < _E // _BE
    eac_p = jnp.where(live, eac_p, 0.0)
    rho_p = jnp.where(live, rho_p, 0.0)
    out_ref[0] = eac_p
    out_ref[1] = rho_p * eac_p
    out_ref[2] = rho_p


def _payload_tc(t4, au, b, W1r, b1r, W2P, b2P):
    return pl.pallas_call(
        _payload_body,
        grid=(_EP // _BE,),
        in_specs=[
            pl.BlockSpec((_BR, 4), lambda i: (i, 0)),
            pl.BlockSpec((_BR, 128), lambda i: (i, 0)),
            pl.BlockSpec((1, 1), lambda i: (0, 0)),
            pl.BlockSpec((256,), lambda i: (0,)),
            pl.BlockSpec((256,), lambda i: (0,)),
            pl.BlockSpec((256, 128), lambda i: (0, 0)),
            pl.BlockSpec((128,), lambda i: (0,)),
        ],
        out_specs=pl.BlockSpec((3, _BR, 128), lambda i: (0, i, 0)),
        out_shape=jax.ShapeDtypeStruct((3, _EP // 4, 128), jnp.float32),
    )(t4, au, b, W1r, b1r, W2P, b2P)


_RPT = _N // 16


@functools.partial(
    pl.kernel,
    out_type=jax.ShapeDtypeStruct((4, _N, 32), jnp.float32),
    mesh=_mesh,
    compiler_params=_sc_params,
    scratch_types=[
        pltpu.VMEM((2, _WS // _IW, 1, _IW), jnp.int32),
        pltpu.VMEM((2, _WS, 32), jnp.float32),
        pltpu.VMEM_SHARED((_N, 32), jnp.float32),
        pltpu.SemaphoreType.DMA,
        pltpu.SemaphoreType.DMA,
        pltpu.SemaphoreType.DMA,
        pltpu.SemaphoreType.DMA,
    ],
)
def _scatter_sc(src_hbm, p_hbm, z_hbm, out_hbm, idx_v, upd_v, acc_sh,
                sd0, sd1, sg0, sg1):
    sd = (sd0, sd1)
    sg = (sg0, sg1)
    c = lax.axis_index("c")
    s = lax.axis_index("s")
    r0 = s * _RPT

    def run_phase(slab, out_slab, ebase, ecount):
        nwin = ecount // _WS
        pltpu.sync_copy(z_hbm.at[pl.ds(r0, _RPT)], acc_sh.at[pl.ds(r0, _RPT)])
        plsc.subcore_barrier()

        def dma_slices(w):
            base = ebase + s * ecount + w * _WS
            return (src_hbm.at[pl.ds(base // _IW, _WS // _IW)],
                    p_hbm.at[slab, pl.ds(base, _WS)])

        def start_dma(w, b):
            ss, pp = dma_slices(w)
            pltpu.async_copy(ss, idx_v.at[b], sd[b])
            pltpu.async_copy(pp, upd_v.at[b], sd[b])

        def wait_dma(w, b):
            ss, pp = dma_slices(w)
            pltpu.make_async_copy(ss, idx_v.at[b], sd[b]).wait()
            pltpu.make_async_copy(pp, upd_v.at[b], sd[b]).wait()

        start_dma(0, 0)
        start_dma(1, 1)

        def stage(w, b):
            wait_dma(w, b)
            cps = []
            for j in range(_WS // _IW):
                cps.append(pltpu.async_copy(
                    upd_v.at[b, pl.ds(j * _IW, _IW)],
                    acc_sh.at[idx_v.at[b, j, 0]], sg[b], add=True))
            for cp in cps:
                cp.wait()

            @pl.when(w + 2 < nwin)
            def _():
                start_dma(w + 2, b)

        def body2(h, carry):
            stage(2 * h, 0)
            stage(2 * h + 1, 1)
            return carry

        lax.fori_loop(0, nwin // 2, body2, 0)
        plsc.subcore_barrier()
        pltpu.sync_copy(acc_sh.at[pl.ds(r0, _RPT)],
                        out_hbm.at[out_slab, pl.ds(r0, _RPT)])
        plsc.subcore_barrier()

    run_phase(c, c, 0, _EPT)
    run_phase(2, 2 + c, c * (_EP // 2), _EP // 32)


def kernel(x, edge_attr, a, b, gamma1, gamma2, bias, W1, b1, W2, b2, edge_index):
    x0 = x[:, 0, :]
    x0p = jnp.pad(x0, ((0, 0), (0, 32 - _H)))
    pad = _EP - _E
    srcp = jnp.concatenate([edge_index[0], jnp.zeros((pad,), jnp.int32)])
    dstp = jnp.concatenate([edge_index[1], jnp.zeros((pad,), jnp.int32)])
    eap = jnp.concatenate([edge_attr[:, 0], jnp.zeros((pad,), jnp.float32)])
    src3 = srcp.reshape(_EP // _IW, 1, _IW)
    dst3 = dstp.reshape(_EP // _IW, 1, _IW)
    av = jnp.full((16,), a[0], jnp.float32)

    W1r = jnp.tile(W1[0], 4)
    b1r = jnp.tile(b1, 4)
    lane = jnp.arange(128)
    hid = jnp.arange(256)
    col = lane % 32
    grp = lane // 32
    sel = (hid[:, None] // _EH == grp[None, :]) & (col[None, :] >= 10) & (
        col[None, :] < 20)
    W2P = jnp.where(sel, W2[hid % _EH][:, jnp.clip(col - 10, 0, 9)], 0.0)
    b2P = jnp.where((col >= 10) & (col < 20), b2[jnp.clip(col - 10, 0, 9)], 0.0)

    au = _gather_sc(x0p, src3, dst3, av)
    t4 = eap.reshape(_EP // 4, 4)
    p = _payload_tc(t4, au, b.reshape(1, 1), W1r, b1r, W2P, b2P)
    p = p.reshape(3, _EP, 32)
    zeros = jnp.zeros((_N, 32), jnp.float32)
    accs = _scatter_sc(src3, p, zeros)
    denom = accs[0, :, :_H]
    s1 = accs[1, :, :_H]
    s0 = (accs[2] + accs[3])[:, :_H]
    sf = jnp.where(denom != 0, s1 / jnp.where(denom != 0, denom, 1.0), 0.01 * s0)
    out0 = jax.nn.sigmoid(x0 @ gamma1.T + sf @ gamma2.T + bias)
    return jnp.stack([out0, sf], axis=1)
